# baseline (speedup 1.0000x reference)
import functools

import jax
import jax.numpy as jnp
from jax import lax
from jax.experimental import pallas as pl
from jax.experimental.pallas import tpu as pltpu
from jax.experimental.pallas import tpu_sc as plsc

D = 128
B = 16384 * 50
NC = 2
NS = 16
NW = NC * NS
BPW = B // NW
CH = 128
NCH = BPW // CH
NBUF = 6

_mesh = plsc.VectorSubcoreMesh(core_axis_name="c", subcore_axis_name="s")


@functools.partial(
    pl.kernel,
    mesh=_mesh,
    out_type=jax.ShapeDtypeStruct((NW, NCH, CH, D), jnp.float32),
    scratch_types=[
        pltpu.VMEM((NCH, CH), jnp.int32),
        pltpu.VMEM((NBUF, CH, D), jnp.float32),
        pltpu.VMEM_SHARED((NS, 5, D), jnp.float32),
        pltpu.SemaphoreType.DMA((NBUF,)),
        pltpu.SemaphoreType.DMA((NBUF,)),
    ],
)
def _emb_lookup(idx_hbm, table_hbm, out_hbm, idx_v, buf_v, table_v, gsem, ssem):
    cid = lax.axis_index("c")
    sid = lax.axis_index("s")
    wid = sid * NC + cid
    pltpu.sync_copy(table_hbm, table_v.at[sid, pl.ds(0, 4)])
    pltpu.sync_copy(idx_hbm.at[wid], idx_v)
    plsc.subcore_barrier()
    gat = [None] * NBUF
    scat = [None] * NBUF
    for c in range(NCH + 1):
        if c < NCH:
            b = c % NBUF
            if scat[b] is not None:
                scat[b].wait()
            gat[b] = pltpu.async_copy(
                table_v.at[sid].at[idx_v.at[c]], buf_v.at[b],
                gsem.at[b])
        if c >= 1:
            pb = (c - 1) % NBUF
            gat[pb].wait()
            scat[pb] = pltpu.async_copy(
                buf_v.at[pb], out_hbm.at[wid, c - 1], ssem.at[pb])
    for b in range(NBUF):
        if scat[b] is not None:
            scat[b].wait()


def kernel(input, embedding_weight):
    idx = input.reshape(NW, NCH, CH).astype(jnp.int32)
    out = _emb_lookup(idx, embedding_weight)
    return out.reshape(1, B, D)


# speedup vs baseline: 1.0042x; 1.0042x over previous
import functools

import jax
import jax.numpy as jnp
from jax import lax
from jax.experimental import pallas as pl
from jax.experimental.pallas import tpu as pltpu
from jax.experimental.pallas import tpu_sc as plsc

D = 128
B = 16384 * 50
NC = 2
NS = 16
NW = NC * NS
BPW = B // NW
CH = 128
NCH = BPW // CH
NBUF = 4

_mesh = plsc.VectorSubcoreMesh(core_axis_name="c", subcore_axis_name="s")


@functools.partial(
    pl.kernel,
    mesh=_mesh,
    out_type=jax.ShapeDtypeStruct((NW, NCH, CH, D), jnp.float32),
    scratch_types=[
        pltpu.VMEM((NCH, CH), jnp.int32),
        pltpu.VMEM((NBUF, CH, D), jnp.float32),
        pltpu.VMEM_SHARED((4, D), jnp.float32),
        pltpu.SemaphoreType.DMA((NBUF,)),
        pltpu.SemaphoreType.DMA((NBUF,)),
    ],
)
def _emb_lookup(idx_hbm, table_hbm, out_hbm, idx_v, buf_v, table_v, gsem, ssem):
    cid = lax.axis_index("c")
    sid = lax.axis_index("s")
    wid = sid * NC + cid
    @pl.when(sid == 0)
    def _():
        pltpu.sync_copy(table_hbm, table_v)

    pltpu.sync_copy(idx_hbm.at[wid], idx_v)
    plsc.subcore_barrier()
    gat = [None] * NBUF
    scat = [None] * NBUF
    for c in range(NCH + 1):
        if c < NCH:
            b = c % NBUF
            if scat[b] is not None:
                scat[b].wait()
            gat[b] = pltpu.async_copy(
                table_v.at[idx_v.at[c]], buf_v.at[b], gsem.at[b])
        if c >= 1:
            pb = (c - 1) % NBUF
            gat[pb].wait()
            scat[pb] = pltpu.async_copy(
                buf_v.at[pb], out_hbm.at[wid, c - 1], ssem.at[pb])
    for b in range(NBUF):
        if scat[b] is not None:
            scat[b].wait()


def kernel(input, embedding_weight):
    idx = input.reshape(NW, NCH, CH).astype(jnp.int32)
    out = _emb_lookup(idx, embedding_weight)
    return out.reshape(1, B, D)
